# baseline (speedup 1.0000x reference)
import functools

import jax
import jax.numpy as jnp
from jax import lax
from jax.experimental import pallas as pl
from jax.experimental.pallas import tpu as pltpu
from jax.experimental.pallas import tpu_sc as plsc

B, S, D, E, C = 4, 2048, 4096, 64, 32

_S_TILE = 256
_DP_ROWS = 16

_NC, _NS, _L = 2, 16, 16
_NW = _NC * _NS
_PAIR = 2
_RPW = _PAIR * E // _NW
_CHUNKS = S // _L


def _gate_body(x_ref, r_ref, out_ref):
    xt = x_ref[0]
    logits = jax.lax.dot_general(
        r_ref[...], xt, (((1,), (1,)), ((), ())),
        preferred_element_type=jnp.float32)
    m = jnp.max(logits, axis=0, keepdims=True)
    u = jnp.exp(logits - m)
    z = jnp.sum(u, axis=0, keepdims=True)
    out_ref[0] = u / z


def _sc_topk(aff, gat, ind, row_v, gbuf, ibuf):
    wid = lax.axis_index("s") * _NC + lax.axis_index("c")
    base = wid * _RPW
    iota = lax.iota(jnp.int32, _L)
    neg_inf = jnp.float32(-jnp.inf)

    def _row(r, c):
        row = base + r
        pltpu.sync_copy(aff.at[pl.ds(pl.multiple_of(row * S, S), S)], row_v)

        def _chunk(j, carry):
            _, _, clo_v, _ = carry
            v = row_v[pl.ds(j * _L, _L)]
            rv, ri = plsc.sort_key_val(v, iota + j * _L, descending=True)

            def _merge(args):
                hi_v, hi_i, lo_v, lo_i = args
                m1 = lo_v < rv
                rh, rhi = plsc.sort_key_val(jnp.where(m1, rv, lo_v),
                                            jnp.where(m1, ri, lo_i),
                                            descending=True)
                m2 = hi_v < rh
                hi_n, hi_ni = plsc.sort_key_val(jnp.where(m2, rh, hi_v),
                                                jnp.where(m2, rhi, hi_i))
                lo_n, lo_ni = plsc.sort_key_val(jnp.where(m2, hi_v, rh),
                                                jnp.where(m2, hi_i, rhi))
                return hi_n, hi_ni, lo_n, lo_ni

            return lax.cond(rv[0] > clo_v[0], _merge, lambda a: a, carry)

        init = (jnp.full((_L,), neg_inf), jnp.zeros((_L,), jnp.int32),
                jnp.full((_L,), neg_inf), jnp.zeros((_L,), jnp.int32))
        hi_v, hi_i, lo_v, lo_i = lax.fori_loop(0, _CHUNKS, _chunk, init,
                                               unroll=4)

        g_hi, i_hi = plsc.sort_key_val(hi_v, hi_i, descending=True)
        g_lo, i_lo = plsc.sort_key_val(lo_v, lo_i, descending=True)
        gbuf[pl.ds(0, _L)] = g_hi
        gbuf[pl.ds(_L, _L)] = g_lo
        ibuf[pl.ds(0, _L)] = i_hi
        ibuf[pl.ds(_L, _L)] = i_lo
        goff = pl.multiple_of(row * C, C)
        pltpu.sync_copy(gbuf, gat.at[pl.ds(goff, C)])
        pltpu.sync_copy(ibuf, ind.at[pl.ds(goff, C)])
        return c

    lax.fori_loop(0, _RPW, _row, 0)


def _dispatch_body(i_ref, d_ref):
    idx = i_ref[...]
    d_ref[...] = (
        idx[:, :, None]
        == jax.lax.broadcasted_iota(jnp.int32, (_DP_ROWS, C, S), 2)
    ).astype(jnp.float32)


@jax.jit
def kernel(x, router):
    sc_call = pl.kernel(
        _sc_topk,
        out_type=[
            jax.ShapeDtypeStruct((_PAIR * E * C,), jnp.float32),
            jax.ShapeDtypeStruct((_PAIR * E * C,), jnp.int32),
        ],
        mesh=plsc.VectorSubcoreMesh(core_axis_name="c", subcore_axis_name="s"),
        compiler_params=pltpu.CompilerParams(needs_layout_passes=False),
        scratch_types=[
            pltpu.VMEM((S,), jnp.float32),
            pltpu.VMEM((C,), jnp.float32),
            pltpu.VMEM((C,), jnp.int32),
        ],
    )

    gats, inds = [], []
    for p in range(B // _PAIR):
        aff_p = pl.pallas_call(
            _gate_body,
            grid=(_PAIR, S // _S_TILE),
            in_specs=[
                pl.BlockSpec((1, _S_TILE, D),
                             lambda bb, s, p=p: (p * _PAIR + bb, s, 0)),
                pl.BlockSpec((E, D), lambda bb, s: (0, 0)),
            ],
            out_specs=pl.BlockSpec((1, E, _S_TILE), lambda bb, s: (bb, 0, s)),
            out_shape=jax.ShapeDtypeStruct((_PAIR, E, S), jnp.float32),
        )(x, router)
        g_p, i_p = sc_call(aff_p.reshape(_PAIR * E * S))
        gats.append(g_p)
        inds.append(i_p)

    gating = jnp.concatenate(gats)
    index = jnp.concatenate(inds)

    index2 = index.reshape(B * E, C)
    dispatch = pl.pallas_call(
        _dispatch_body,
        grid=(B * E // _DP_ROWS,),
        in_specs=[pl.BlockSpec((_DP_ROWS, C), lambda i: (i, 0))],
        out_specs=pl.BlockSpec((_DP_ROWS, C, S), lambda i: (i, 0, 0)),
        out_shape=jax.ShapeDtypeStruct((B * E, C, S), jnp.float32),
    )(index2)

    return (gating.reshape(B, E, C),
            dispatch.reshape(B, E, C, S),
            index.reshape(B, E, C))


# speedup vs baseline: 1.0629x; 1.0629x over previous
import functools

import jax
import jax.numpy as jnp
from jax import lax
from jax.experimental import pallas as pl
from jax.experimental.pallas import tpu as pltpu
from jax.experimental.pallas import tpu_sc as plsc

B, S, D, E, C = 4, 2048, 4096, 64, 32

_S_TILE = 512
_DP_ROWS = 16

_NC, _NS, _L = 2, 16, 16
_NW = _NC * _NS
_PAIR = 2
_RPW = _PAIR * E // _NW
_CHUNKS = S // _L


def _gate_body(x_ref, r_ref, out_ref):
    xt = x_ref[0]
    logits = jax.lax.dot_general(
        r_ref[...], xt, (((1,), (1,)), ((), ())),
        preferred_element_type=jnp.float32)
    m = jnp.max(logits, axis=0, keepdims=True)
    u = jnp.exp(logits - m)
    z = jnp.sum(u, axis=0, keepdims=True)
    out_ref[0] = u / z


def _sc_topk(aff, gat, ind, row_v, gbuf, ibuf):
    wid = lax.axis_index("s") * _NC + lax.axis_index("c")
    base = wid * _RPW
    iota = lax.iota(jnp.int32, _L)
    neg_inf = jnp.float32(-jnp.inf)

    def _row(r, c):
        row = base + r
        pltpu.sync_copy(aff.at[pl.ds(pl.multiple_of(row * S, S), S)], row_v)

        def _chunk(j, carry):
            _, _, clo_v, _ = carry
            v = row_v[pl.ds(j * _L, _L)]
            rv, ri = plsc.sort_key_val(v, iota + j * _L, descending=True)

            def _merge(args):
                hi_v, hi_i, lo_v, lo_i = args
                m1 = lo_v < rv
                rh, rhi = plsc.sort_key_val(jnp.where(m1, rv, lo_v),
                                            jnp.where(m1, ri, lo_i),
                                            descending=True)
                m2 = hi_v < rh
                hi_n, hi_ni = plsc.sort_key_val(jnp.where(m2, rh, hi_v),
                                                jnp.where(m2, rhi, hi_i))
                lo_n, lo_ni = plsc.sort_key_val(jnp.where(m2, hi_v, rh),
                                                jnp.where(m2, hi_i, rhi))
                return hi_n, hi_ni, lo_n, lo_ni

            return lax.cond(rv[0] > clo_v[0], _merge, lambda a: a, carry)

        init = (jnp.full((_L,), neg_inf), jnp.zeros((_L,), jnp.int32),
                jnp.full((_L,), neg_inf), jnp.zeros((_L,), jnp.int32))
        hi_v, hi_i, lo_v, lo_i = lax.fori_loop(0, _CHUNKS, _chunk, init,
                                               unroll=4)

        g_hi, i_hi = plsc.sort_key_val(hi_v, hi_i, descending=True)
        g_lo, i_lo = plsc.sort_key_val(lo_v, lo_i, descending=True)
        gbuf[pl.ds(0, _L)] = g_hi
        gbuf[pl.ds(_L, _L)] = g_lo
        ibuf[pl.ds(0, _L)] = i_hi
        ibuf[pl.ds(_L, _L)] = i_lo
        goff = pl.multiple_of(row * C, C)
        pltpu.sync_copy(gbuf, gat.at[pl.ds(goff, C)])
        pltpu.sync_copy(ibuf, ind.at[pl.ds(goff, C)])
        return c

    lax.fori_loop(0, _RPW, _row, 0)


def _dispatch_body(i_ref, d_ref):
    idx = i_ref[...]
    d_ref[...] = (
        idx[:, :, None]
        == jax.lax.broadcasted_iota(jnp.int32, (_DP_ROWS, C, S), 2)
    ).astype(jnp.float32)


@jax.jit
def kernel(x, router):
    sc_call = pl.kernel(
        _sc_topk,
        out_type=[
            jax.ShapeDtypeStruct((_PAIR * E * C,), jnp.float32),
            jax.ShapeDtypeStruct((_PAIR * E * C,), jnp.int32),
        ],
        mesh=plsc.VectorSubcoreMesh(core_axis_name="c", subcore_axis_name="s"),
        compiler_params=pltpu.CompilerParams(needs_layout_passes=False),
        scratch_types=[
            pltpu.VMEM((S,), jnp.float32),
            pltpu.VMEM((C,), jnp.float32),
            pltpu.VMEM((C,), jnp.int32),
        ],
    )

    gats, inds = [], []
    for p in range(B // _PAIR):
        aff_p = pl.pallas_call(
            _gate_body,
            grid=(_PAIR, S // _S_TILE),
            in_specs=[
                pl.BlockSpec((1, _S_TILE, D),
                             lambda bb, s, p=p: (p * _PAIR + bb, s, 0)),
                pl.BlockSpec((E, D), lambda bb, s: (0, 0)),
            ],
            out_specs=pl.BlockSpec((1, E, _S_TILE), lambda bb, s: (bb, 0, s)),
            out_shape=jax.ShapeDtypeStruct((_PAIR, E, S), jnp.float32),
        )(x, router)
        g_p, i_p = sc_call(aff_p.reshape(_PAIR * E * S))
        gats.append(g_p)
        inds.append(i_p)

    gating = jnp.concatenate(gats)
    index = jnp.concatenate(inds)

    index2 = index.reshape(B * E, C)
    dispatch = pl.pallas_call(
        _dispatch_body,
        grid=(B * E // _DP_ROWS,),
        in_specs=[pl.BlockSpec((_DP_ROWS, C), lambda i: (i, 0))],
        out_specs=pl.BlockSpec((_DP_ROWS, C, S), lambda i: (i, 0, 0)),
        out_shape=jax.ShapeDtypeStruct((B * E, C, S), jnp.float32),
    )(index2)

    return (gating.reshape(B, E, C),
            dispatch.reshape(B, E, C, S),
            index.reshape(B, E, C))
